# baseline (speedup 1.0000x reference)
import functools
import math

import jax
import jax.numpy as jnp
from jax import lax
from jax.experimental import pallas as pl
from jax.experimental.pallas import tpu as pltpu
from jax.experimental.pallas import tpu_sc as plsc

N_NODES = 10000
N_EDGES = 320000
NODE_DIM = 128
EDGE_DIM = 16
N_HEADS = 8
D_K = NODE_DIM // N_HEADS
INV_SQRT_DK = 1.0 / math.sqrt(D_K)

N_PAD = 10240
NUM_TILES = 32
CHUNK = 128
CHUNKS_PER_TILE = 80
E_PAD = NUM_TILES * CHUNKS_PER_TILE * CHUNK
SCHUNK = 128
SCHUNKS_PER_TILE = E_PAD // (NUM_TILES * SCHUNK)
ROWS_PER_TILE = N_PAD // 16

_HI = jax.lax.Precision.HIGHEST


def _qkv_body(x, wq, bq, wk, bk, wv, bv, q_o, k_o, v_o):
    xv = x[...]
    q_o[...] = jnp.dot(xv, wq[...], preferred_element_type=jnp.float32,
                       precision=_HI) + bq[...]
    k_o[...] = jnp.dot(xv, wk[...], preferred_element_type=jnp.float32,
                       precision=_HI) + bk[...]
    v_o[...] = jnp.dot(xv, wv[...], preferred_element_type=jnp.float32,
                       precision=_HI) + bv[...]


def _qkv(nodes_pad, Wq, bq, Wk, bk, Wv, bv):
    blk = 1024
    grid = N_PAD // blk
    full = pl.BlockSpec((NODE_DIM, NODE_DIM), lambda i: (0, 0))
    bias = pl.BlockSpec((1, NODE_DIM), lambda i: (0, 0))
    nb = pl.BlockSpec((blk, NODE_DIM), lambda i: (i, 0))
    return pl.pallas_call(
        _qkv_body,
        grid=(grid,),
        in_specs=[nb, full, bias, full, bias, full, bias],
        out_specs=[nb, nb, nb],
        out_shape=[jax.ShapeDtypeStruct((N_PAD, NODE_DIM), jnp.float32)] * 3,
    )(nodes_pad, Wq, bq.reshape(1, -1), Wk, bk.reshape(1, -1),
      Wv, bv.reshape(1, -1))


def _gather_kernel(q_hbm, k_hbm, v_hbm, ridx_hbm, cidx_hbm,
                   qg_hbm, kg_hbm, vg_hbm,
                   ridx_v, cidx_v, qrows, krows, vrows, sem0, sem1, sem2):
    g = lax.axis_index("c") * 16 + lax.axis_index("s")
    c0 = g * CHUNKS_PER_TILE
    pltpu.sync_copy(ridx_hbm.at[pl.ds(c0, CHUNKS_PER_TILE)], ridx_v)
    pltpu.sync_copy(cidx_hbm.at[pl.ds(c0, CHUNKS_PER_TILE)], cidx_v)

    @pl.loop(0, CHUNKS_PER_TILE)
    def _(i):
        base = (c0 + i) * CHUNK
        cq = pltpu.async_copy(q_hbm.at[ridx_v.at[i]], qrows, sem0)
        ck = pltpu.async_copy(k_hbm.at[cidx_v.at[i]], krows, sem1)
        cv = pltpu.async_copy(v_hbm.at[cidx_v.at[i]], vrows, sem2)
        cq.wait()
        ck.wait()
        cv.wait()
        pltpu.sync_copy(qrows, qg_hbm.at[pl.ds(base, CHUNK)])
        pltpu.sync_copy(krows, kg_hbm.at[pl.ds(base, CHUNK)])
        pltpu.sync_copy(vrows, vg_hbm.at[pl.ds(base, CHUNK)])


def _gather(q, k, v, ridx, cidx):
    mesh = plsc.VectorSubcoreMesh(core_axis_name="c", subcore_axis_name="s")
    kern = pl.kernel(
        _gather_kernel,
        mesh=mesh,
        out_type=[jax.ShapeDtypeStruct((E_PAD, NODE_DIM), jnp.float32)] * 3,
        scratch_types=[
            pltpu.VMEM((CHUNKS_PER_TILE, CHUNK), jnp.int32),
            pltpu.VMEM((CHUNKS_PER_TILE, CHUNK), jnp.int32),
            pltpu.VMEM((CHUNK, NODE_DIM), jnp.float32),
            pltpu.VMEM((CHUNK, NODE_DIM), jnp.float32),
            pltpu.VMEM((CHUNK, NODE_DIM), jnp.float32),
            pltpu.SemaphoreType.DMA,
            pltpu.SemaphoreType.DMA,
            pltpu.SemaphoreType.DMA,
        ],
    )
    return kern(q, k, v, ridx, cidx)


def _edge_w_body(qg, kg, ea, we, be, w_o):
    ef = jnp.dot(ea[...], we[...], preferred_element_type=jnp.float32,
                 precision=_HI) + be[...]
    t = qg[...] * (kg[...] + ef)
    r = lax.broadcasted_iota(jnp.int32, (NODE_DIM, NODE_DIM), 0)
    c = lax.broadcasted_iota(jnp.int32, (NODE_DIM, NODE_DIM), 1)
    e8 = (r // D_K == c // D_K).astype(jnp.float32)
    scores = jnp.dot(t, e8, preferred_element_type=jnp.float32,
                     precision=_HI) * INV_SQRT_DK
    w_o[...] = jnp.exp(scores)


def _edge_w(qg, kg, ea_pad, We, be):
    blk = 1024
    grid = E_PAD // blk
    eb = pl.BlockSpec((blk, NODE_DIM), lambda i: (i, 0))
    return pl.pallas_call(
        _edge_w_body,
        grid=(grid,),
        in_specs=[eb, eb,
                  pl.BlockSpec((blk, EDGE_DIM), lambda i: (i, 0)),
                  pl.BlockSpec((EDGE_DIM, NODE_DIM), lambda i: (0, 0)),
                  pl.BlockSpec((1, NODE_DIM), lambda i: (0, 0))],
        out_specs=eb,
        out_shape=jax.ShapeDtypeStruct((E_PAD, NODE_DIM), jnp.float32),
    )(qg, kg, ea_pad, We, be.reshape(1, -1))


def _edge_msg_body(w, vg, msg_o):
    msg_o[...] = w[...] * vg[...]


def _edge_msg(w, vg):
    blk = 1024
    grid = E_PAD // blk
    eb = pl.BlockSpec((blk, NODE_DIM), lambda i: (i, 0))
    return pl.pallas_call(
        _edge_msg_body,
        grid=(grid,),
        in_specs=[eb, eb],
        out_specs=eb,
        out_shape=jax.ShapeDtypeStruct((E_PAD, NODE_DIM), jnp.float32),
    )(w, vg)


def _scatter_kernel(msg_hbm, ridx_hbm, z_hbm, acc_out,
                    idx0, buf0, shared_acc, sem):
    cid = lax.axis_index("c")
    sid = lax.axis_index("s")
    g = cid * 16 + sid
    r0 = sid * ROWS_PER_TILE
    iota = lax.iota(jnp.int32, 16)

    pltpu.sync_copy(z_hbm, buf0.at[pl.ds(0, 64)])

    @pl.loop(0, ROWS_PER_TILE // 16)
    def _(j):
        idxv = r0 + j * 16 + iota
        pltpu.sync_copy(buf0.at[pl.ds(0, 16)], shared_acc.at[idxv])

    plsc.subcore_barrier()

    c0 = g * SCHUNKS_PER_TILE
    pltpu.sync_copy(ridx_hbm.at[pl.ds(c0, SCHUNKS_PER_TILE)], idx0)

    @pl.loop(0, SCHUNKS_PER_TILE)
    def _(i):
        cm = pltpu.async_copy(msg_hbm.at[pl.ds((c0 + i) * SCHUNK, SCHUNK)],
                              buf0, sem)
        cm.wait()
        pltpu.sync_copy(buf0, shared_acc.at[idx0.at[i, 0]], add=True)

    plsc.subcore_barrier()

    @pl.loop(0, ROWS_PER_TILE // 16)
    def _(j):
        rr = r0 + j * 16
        idxv = rr + iota
        pltpu.sync_copy(shared_acc.at[idxv], buf0.at[pl.ds(0, 16)])
        pltpu.sync_copy(buf0.at[pl.ds(0, 16)],
                        acc_out.at[pl.ds(cid * N_PAD + rr, 16)])


def _scatter(msg, ridx, z):
    mesh = plsc.VectorSubcoreMesh(core_axis_name="c", subcore_axis_name="s")
    kern = pl.kernel(
        _scatter_kernel,
        mesh=mesh,
        out_type=jax.ShapeDtypeStruct((2 * N_PAD, NODE_DIM), jnp.float32),
        scratch_types=[
            pltpu.VMEM((SCHUNKS_PER_TILE, 1, SCHUNK), jnp.int32),
            pltpu.VMEM((SCHUNK, NODE_DIM), jnp.float32),
            pltpu.VMEM_SHARED((N_PAD, NODE_DIM), jnp.float32),
            pltpu.SemaphoreType.DMA,
        ],
    )
    return kern(msg, ridx.reshape(-1, 1, SCHUNK), z)


def _final_body(num, den, wo, bo, out):
    n = num[0] + num[1]
    d = den[0] + den[1]
    out[...] = jnp.dot(n * (1.0 / (d + 1e-8)), wo[...],
                       preferred_element_type=jnp.float32,
                       precision=_HI) + bo[...]


def _finalize(num_parts, den_parts, Wo, bo):
    blk = 1000
    grid = N_NODES // blk
    nb = pl.BlockSpec((2, blk, NODE_DIM), lambda i: (0, i, 0))
    return pl.pallas_call(
        _final_body,
        grid=(grid,),
        in_specs=[nb, nb,
                  pl.BlockSpec((NODE_DIM, NODE_DIM), lambda i: (0, 0)),
                  pl.BlockSpec((1, NODE_DIM), lambda i: (0, 0))],
        out_specs=pl.BlockSpec((blk, NODE_DIM), lambda i: (i, 0)),
        out_shape=jax.ShapeDtypeStruct((N_NODES, NODE_DIM), jnp.float32),
    )(num_parts, den_parts, Wo, bo.reshape(1, -1))


def kernel(nodes, edge_index, edge_attr, Wq, bq, Wk, bk, Wv, bv, We, be,
           Wo, bo):
    nodes_pad = jnp.pad(nodes, ((0, N_PAD - N_NODES), (0, 0)))
    q, k, v = _qkv(nodes_pad, Wq, bq, Wk, bk, Wv, bv)

    ea_pad = jnp.pad(edge_attr, ((0, E_PAD - N_EDGES), (0, 0)))

    row = edge_index[0].astype(jnp.int32)
    col = edge_index[1].astype(jnp.int32)
    pad_i = N_NODES + jnp.arange(E_PAD - N_EDGES, dtype=jnp.int32) % (
        N_PAD - N_NODES)
    ridx = jnp.concatenate([row, pad_i]).reshape(-1, CHUNK)
    cidx = jnp.concatenate([col, pad_i]).reshape(-1, CHUNK)

    qg, kg, vg = _gather(q, k, v, ridx, cidx)
    w = _edge_w(qg, kg, ea_pad, We, be)

    z = jnp.zeros((64, NODE_DIM), jnp.float32)
    den_parts = _scatter(w, ridx, z)
    msg = _edge_msg(w, vg)
    num_parts = _scatter(msg, ridx, z)

    return _finalize(num_parts.reshape(2, N_PAD, NODE_DIM),
                     den_parts.reshape(2, N_PAD, NODE_DIM), Wo, bo)


# speedup vs baseline: 1.1486x; 1.1486x over previous
import functools
import math

import jax
import jax.numpy as jnp
from jax import lax
from jax.experimental import pallas as pl
from jax.experimental.pallas import tpu as pltpu
from jax.experimental.pallas import tpu_sc as plsc

N_NODES = 10000
N_EDGES = 320000
NODE_DIM = 128
EDGE_DIM = 16
N_HEADS = 8
D_K = NODE_DIM // N_HEADS
INV_SQRT_DK = 1.0 / math.sqrt(D_K)

N_PAD = 10240
NUM_TILES = 32
CHUNK = 128
CHUNKS_PER_TILE = 80
E_PAD = NUM_TILES * CHUNKS_PER_TILE * CHUNK
SCHUNK = 128
SCHUNKS_PER_TILE = E_PAD // (NUM_TILES * SCHUNK)
ROWS_PER_TILE = N_PAD // 16

_HI = jax.lax.Precision.HIGHEST


def _qkv_body(x, wq, bq, wk, bk, wv, bv, q_o, k_o, v_o):
    xv = x[...]
    q_o[...] = jnp.dot(xv, wq[...], preferred_element_type=jnp.float32,
                       precision=_HI) + bq[...]
    k_o[...] = jnp.dot(xv, wk[...], preferred_element_type=jnp.float32,
                       precision=_HI) + bk[...]
    v_o[...] = jnp.dot(xv, wv[...], preferred_element_type=jnp.float32,
                       precision=_HI) + bv[...]


def _qkv(nodes_pad, Wq, bq, Wk, bk, Wv, bv):
    blk = 1024
    grid = N_PAD // blk
    full = pl.BlockSpec((NODE_DIM, NODE_DIM), lambda i: (0, 0))
    bias = pl.BlockSpec((1, NODE_DIM), lambda i: (0, 0))
    nb = pl.BlockSpec((blk, NODE_DIM), lambda i: (i, 0))
    return pl.pallas_call(
        _qkv_body,
        grid=(grid,),
        in_specs=[nb, full, bias, full, bias, full, bias],
        out_specs=[nb, nb, nb],
        out_shape=[jax.ShapeDtypeStruct((N_PAD, NODE_DIM), jnp.float32)] * 3,
    )(nodes_pad, Wq, bq.reshape(1, -1), Wk, bk.reshape(1, -1),
      Wv, bv.reshape(1, -1))


def _gather_kernel(q_hbm, k_hbm, v_hbm, ridx_hbm, cidx_hbm,
                   qg_hbm, kg_hbm, vg_hbm,
                   ridx_v, cidx_v, qrows, krows, vrows, sem0, sem1, sem2):
    g = lax.axis_index("c") * 16 + lax.axis_index("s")
    c0 = g * CHUNKS_PER_TILE
    pltpu.sync_copy(ridx_hbm.at[pl.ds(c0, CHUNKS_PER_TILE)], ridx_v)
    pltpu.sync_copy(cidx_hbm.at[pl.ds(c0, CHUNKS_PER_TILE)], cidx_v)

    @pl.loop(0, CHUNKS_PER_TILE)
    def _(i):
        base = (c0 + i) * CHUNK
        cq = pltpu.async_copy(q_hbm.at[ridx_v.at[i]], qrows, sem0)
        ck = pltpu.async_copy(k_hbm.at[cidx_v.at[i]], krows, sem1)
        cv = pltpu.async_copy(v_hbm.at[cidx_v.at[i]], vrows, sem2)
        cq.wait()
        ck.wait()
        cv.wait()
        pltpu.sync_copy(qrows, qg_hbm.at[pl.ds(base, CHUNK)])
        pltpu.sync_copy(krows, kg_hbm.at[pl.ds(base, CHUNK)])
        pltpu.sync_copy(vrows, vg_hbm.at[pl.ds(base, CHUNK)])


def _gather(q, k, v, ridx, cidx):
    mesh = plsc.VectorSubcoreMesh(core_axis_name="c", subcore_axis_name="s")
    kern = pl.kernel(
        _gather_kernel,
        mesh=mesh,
        out_type=[jax.ShapeDtypeStruct((E_PAD, NODE_DIM), jnp.float32)] * 3,
        scratch_types=[
            pltpu.VMEM((CHUNKS_PER_TILE, CHUNK), jnp.int32),
            pltpu.VMEM((CHUNKS_PER_TILE, CHUNK), jnp.int32),
            pltpu.VMEM((CHUNK, NODE_DIM), jnp.float32),
            pltpu.VMEM((CHUNK, NODE_DIM), jnp.float32),
            pltpu.VMEM((CHUNK, NODE_DIM), jnp.float32),
            pltpu.SemaphoreType.DMA,
            pltpu.SemaphoreType.DMA,
            pltpu.SemaphoreType.DMA,
        ],
    )
    return kern(q, k, v, ridx, cidx)


def _edge_math_body(qg, kg, vg, ea, we, be, msg_o, w_o):
    ef = jnp.dot(ea[...], we[...], preferred_element_type=jnp.float32,
                 precision=_HI) + be[...]
    t = qg[...] * (kg[...] + ef)
    r = lax.broadcasted_iota(jnp.int32, (NODE_DIM, NODE_DIM), 0)
    c = lax.broadcasted_iota(jnp.int32, (NODE_DIM, NODE_DIM), 1)
    e8 = (r // D_K == c // D_K).astype(jnp.float32)
    scores = jnp.dot(t, e8, preferred_element_type=jnp.float32,
                     precision=_HI) * INV_SQRT_DK
    w = jnp.exp(scores)
    msg_o[...] = w * vg[...]
    w_o[...] = w


def _edge_math(qg, kg, vg, ea_pad, We, be):
    blk = 1024
    grid = E_PAD // blk
    eb = pl.BlockSpec((blk, NODE_DIM), lambda i: (i, 0))
    return pl.pallas_call(
        _edge_math_body,
        grid=(grid,),
        in_specs=[eb, eb, eb,
                  pl.BlockSpec((blk, EDGE_DIM), lambda i: (i, 0)),
                  pl.BlockSpec((EDGE_DIM, NODE_DIM), lambda i: (0, 0)),
                  pl.BlockSpec((1, NODE_DIM), lambda i: (0, 0))],
        out_specs=[eb, eb],
        out_shape=[jax.ShapeDtypeStruct((E_PAD, NODE_DIM), jnp.float32),
                   jax.ShapeDtypeStruct((E_PAD, NODE_DIM), jnp.float32)],
    )(qg, kg, vg, ea_pad, We, be.reshape(1, -1))


def _scatter_kernel(msg_hbm, w_hbm, ridx_hbm, z_hbm, num_out, den_out,
                    idx0, buf0, shared_acc, sem):
    cid = lax.axis_index("c")
    sid = lax.axis_index("s")
    g = cid * 16 + sid
    r0 = sid * ROWS_PER_TILE
    iota = lax.iota(jnp.int32, 16)
    c0 = g * SCHUNKS_PER_TILE
    pltpu.sync_copy(ridx_hbm.at[pl.ds(c0, SCHUNKS_PER_TILE)], idx0)

    for src_hbm, out_hbm in ((msg_hbm, num_out), (w_hbm, den_out)):
        pltpu.sync_copy(z_hbm, buf0.at[pl.ds(0, 64)])

        @pl.loop(0, ROWS_PER_TILE // 16)
        def _(j):
            idxv = r0 + j * 16 + iota
            pltpu.sync_copy(buf0.at[pl.ds(0, 16)], shared_acc.at[idxv])

        plsc.subcore_barrier()

        @pl.loop(0, SCHUNKS_PER_TILE)
        def _(i):
            cm = pltpu.async_copy(
                src_hbm.at[pl.ds((c0 + i) * SCHUNK, SCHUNK)], buf0, sem)
            cm.wait()
            pltpu.sync_copy(buf0, shared_acc.at[idx0.at[i, 0]], add=True)

        plsc.subcore_barrier()

        @pl.loop(0, ROWS_PER_TILE // 16)
        def _(j):
            rr = r0 + j * 16
            idxv = rr + iota
            pltpu.sync_copy(shared_acc.at[idxv], buf0.at[pl.ds(0, 16)])
            pltpu.sync_copy(buf0.at[pl.ds(0, 16)],
                            out_hbm.at[pl.ds(cid * N_PAD + rr, 16)])


def _scatter(msg, w, ridx, z):
    mesh = plsc.VectorSubcoreMesh(core_axis_name="c", subcore_axis_name="s")
    kern = pl.kernel(
        _scatter_kernel,
        mesh=mesh,
        out_type=[jax.ShapeDtypeStruct((2 * N_PAD, NODE_DIM), jnp.float32),
                  jax.ShapeDtypeStruct((2 * N_PAD, NODE_DIM), jnp.float32)],
        scratch_types=[
            pltpu.VMEM((SCHUNKS_PER_TILE, 1, SCHUNK), jnp.int32),
            pltpu.VMEM((SCHUNK, NODE_DIM), jnp.float32),
            pltpu.VMEM_SHARED((N_PAD, NODE_DIM), jnp.float32),
            pltpu.SemaphoreType.DMA,
        ],
    )
    return kern(msg, w, ridx.reshape(-1, 1, SCHUNK), z)


def _final_body(num, den, wo, bo, out):
    n = num[0] + num[1]
    d = den[0] + den[1]
    out[...] = jnp.dot(n * (1.0 / (d + 1e-8)), wo[...],
                       preferred_element_type=jnp.float32,
                       precision=_HI) + bo[...]


def _finalize(num_parts, den_parts, Wo, bo):
    blk = 1000
    grid = N_NODES // blk
    nb = pl.BlockSpec((2, blk, NODE_DIM), lambda i: (0, i, 0))
    return pl.pallas_call(
        _final_body,
        grid=(grid,),
        in_specs=[nb, nb,
                  pl.BlockSpec((NODE_DIM, NODE_DIM), lambda i: (0, 0)),
                  pl.BlockSpec((1, NODE_DIM), lambda i: (0, 0))],
        out_specs=pl.BlockSpec((blk, NODE_DIM), lambda i: (i, 0)),
        out_shape=jax.ShapeDtypeStruct((N_NODES, NODE_DIM), jnp.float32),
    )(num_parts, den_parts, Wo, bo.reshape(1, -1))


def kernel(nodes, edge_index, edge_attr, Wq, bq, Wk, bk, Wv, bv, We, be,
           Wo, bo):
    nodes_pad = jnp.pad(nodes, ((0, N_PAD - N_NODES), (0, 0)))
    q, k, v = _qkv(nodes_pad, Wq, bq, Wk, bk, Wv, bv)

    ea_pad = jnp.pad(edge_attr, ((0, E_PAD - N_EDGES), (0, 0)))

    row = edge_index[0].astype(jnp.int32)
    col = edge_index[1].astype(jnp.int32)
    pad_i = N_NODES + jnp.arange(E_PAD - N_EDGES, dtype=jnp.int32) % (
        N_PAD - N_NODES)
    ridx = jnp.concatenate([row, pad_i]).reshape(-1, CHUNK)
    cidx = jnp.concatenate([col, pad_i]).reshape(-1, CHUNK)

    qg, kg, vg = _gather(q, k, v, ridx, cidx)
    msg, w = _edge_math(qg, kg, vg, ea_pad, We, be)

    z = jnp.zeros((64, NODE_DIM), jnp.float32)
    num_parts, den_parts = _scatter(msg, w, ridx, z)

    return _finalize(num_parts.reshape(2, N_PAD, NODE_DIM),
                     den_parts.reshape(2, N_PAD, NODE_DIM), Wo, bo)


# speedup vs baseline: 1.1497x; 1.0009x over previous
import math

import jax
import jax.numpy as jnp
from jax import lax
from jax.experimental import pallas as pl
from jax.experimental.pallas import tpu as pltpu
from jax.experimental.pallas import tpu_sc as plsc

N_NODES = 10000
N_EDGES = 320000
NODE_DIM = 128
EDGE_DIM = 16
N_HEADS = 8
D_K = NODE_DIM // N_HEADS
INV_SQRT_DK = 1.0 / math.sqrt(D_K)

N_PAD = 10240
NUM_TILES = 32
CHUNK = 128
CHUNKS_PER_TILE = 80
E_PAD = NUM_TILES * CHUNKS_PER_TILE * CHUNK
SCHUNK = 128
SCHUNKS_PER_TILE = E_PAD // (NUM_TILES * SCHUNK)
ROWS_PER_TILE = N_PAD // 16

_HI = jax.lax.Precision.HIGHEST


def _qkv_body(x, wq, bq, wk, bk, wv, bv, q_o, k_o, v_o):
    xv = x[...]
    q_o[...] = jnp.dot(xv, wq[...], preferred_element_type=jnp.float32,
                       precision=_HI) + bq[...]
    k_o[...] = jnp.dot(xv, wk[...], preferred_element_type=jnp.float32,
                       precision=_HI) + bk[...]
    v_o[...] = jnp.dot(xv, wv[...], preferred_element_type=jnp.float32,
                       precision=_HI) + bv[...]


def _qkv(nodes_pad, Wq, bq, Wk, bk, Wv, bv):
    blk = 1024
    grid = N_PAD // blk
    full = pl.BlockSpec((NODE_DIM, NODE_DIM), lambda i: (0, 0))
    bias = pl.BlockSpec((1, NODE_DIM), lambda i: (0, 0))
    nb = pl.BlockSpec((blk, NODE_DIM), lambda i: (i, 0))
    return pl.pallas_call(
        _qkv_body,
        grid=(grid,),
        in_specs=[nb, full, bias, full, bias, full, bias],
        out_specs=[nb, nb, nb],
        out_shape=[jax.ShapeDtypeStruct((N_PAD, NODE_DIM), jnp.float32)] * 3,
    )(nodes_pad, Wq, bq.reshape(1, -1), Wk, bk.reshape(1, -1),
      Wv, bv.reshape(1, -1))


def _gather_kernel(q_hbm, k_hbm, v_hbm, ridx_hbm, cidx_hbm,
                   qg_hbm, kg_hbm, vg_hbm,
                   ridx_v, cidx_v, qrows, krows, vrows, sem0, sem1, sem2):
    g = lax.axis_index("c") * 16 + lax.axis_index("s")
    c0 = g * CHUNKS_PER_TILE
    pltpu.sync_copy(ridx_hbm.at[pl.ds(c0, CHUNKS_PER_TILE)], ridx_v)
    pltpu.sync_copy(cidx_hbm.at[pl.ds(c0, CHUNKS_PER_TILE)], cidx_v)

    @pl.loop(0, CHUNKS_PER_TILE)
    def _(i):
        base = (c0 + i) * CHUNK
        cq = pltpu.async_copy(q_hbm.at[ridx_v.at[i]], qrows, sem0)
        ck = pltpu.async_copy(k_hbm.at[cidx_v.at[i]], krows, sem1)
        cv = pltpu.async_copy(v_hbm.at[cidx_v.at[i]], vrows, sem2)
        cq.wait()
        ck.wait()
        cv.wait()
        pltpu.sync_copy(qrows, qg_hbm.at[pl.ds(base, CHUNK)])
        pltpu.sync_copy(krows, kg_hbm.at[pl.ds(base, CHUNK)])
        pltpu.sync_copy(vrows, vg_hbm.at[pl.ds(base, CHUNK)])


def _gather(q, k, v, ridx, cidx):
    mesh = plsc.VectorSubcoreMesh(core_axis_name="c", subcore_axis_name="s")
    kern = pl.kernel(
        _gather_kernel,
        mesh=mesh,
        out_type=[jax.ShapeDtypeStruct((E_PAD, NODE_DIM), jnp.float32)] * 3,
        scratch_types=[
            pltpu.VMEM((CHUNKS_PER_TILE, CHUNK), jnp.int32),
            pltpu.VMEM((CHUNKS_PER_TILE, CHUNK), jnp.int32),
            pltpu.VMEM((CHUNK, NODE_DIM), jnp.float32),
            pltpu.VMEM((CHUNK, NODE_DIM), jnp.float32),
            pltpu.VMEM((CHUNK, NODE_DIM), jnp.float32),
            pltpu.SemaphoreType.DMA,
            pltpu.SemaphoreType.DMA,
            pltpu.SemaphoreType.DMA,
        ],
    )
    return kern(q, k, v, ridx, cidx)


def _edge_math_body(qg, kg, vg, ea, we, be, msg_o, w_o):
    ef = jnp.dot(ea[...], we[...], preferred_element_type=jnp.float32,
                 precision=_HI) + be[...]
    t = qg[...] * (kg[...] + ef)
    r = lax.broadcasted_iota(jnp.int32, (NODE_DIM, NODE_DIM), 0)
    c = lax.broadcasted_iota(jnp.int32, (NODE_DIM, NODE_DIM), 1)
    e8 = (r // D_K == c // D_K).astype(jnp.float32)
    scores = jnp.dot(t, e8, preferred_element_type=jnp.float32,
                     precision=_HI) * INV_SQRT_DK
    w = jnp.exp(scores)
    msg_o[...] = w * vg[...]
    w_o[...] = w


def _edge_math(qg, kg, vg, ea_pad, We, be):
    blk = 1024
    grid = E_PAD // blk
    eb = pl.BlockSpec((blk, NODE_DIM), lambda i: (i, 0))
    return pl.pallas_call(
        _edge_math_body,
        grid=(grid,),
        in_specs=[eb, eb, eb,
                  pl.BlockSpec((blk, EDGE_DIM), lambda i: (i, 0)),
                  pl.BlockSpec((EDGE_DIM, NODE_DIM), lambda i: (0, 0)),
                  pl.BlockSpec((1, NODE_DIM), lambda i: (0, 0))],
        out_specs=[eb, eb],
        out_shape=[jax.ShapeDtypeStruct((E_PAD, NODE_DIM), jnp.float32),
                   jax.ShapeDtypeStruct((E_PAD, NODE_DIM), jnp.float32)],
    )(qg, kg, vg, ea_pad, We, be.reshape(1, -1))


def _scatter_kernel(msg_hbm, w_hbm, ridx_hbm, z_hbm, num_out, den_out,
                    idx0, buf0, shared_acc, sem):
    cid = lax.axis_index("c")
    sid = lax.axis_index("s")
    g = cid * 16 + sid
    r0 = sid * ROWS_PER_TILE
    iota = lax.iota(jnp.int32, 16)
    c0 = g * SCHUNKS_PER_TILE
    pltpu.sync_copy(ridx_hbm.at[pl.ds(c0, SCHUNKS_PER_TILE)], idx0)

    for src_hbm, out_hbm in ((msg_hbm, num_out), (w_hbm, den_out)):
        pltpu.sync_copy(z_hbm, buf0.at[pl.ds(0, 64)])

        @pl.loop(0, ROWS_PER_TILE // 16)
        def _(j):
            idxv = r0 + j * 16 + iota
            pltpu.sync_copy(buf0.at[pl.ds(0, 16)], shared_acc.at[idxv])

        plsc.subcore_barrier()

        @pl.loop(0, SCHUNKS_PER_TILE)
        def _(i):
            cm = pltpu.async_copy(
                src_hbm.at[pl.ds((c0 + i) * SCHUNK, SCHUNK)], buf0, sem)
            cm.wait()
            pltpu.sync_copy(buf0, shared_acc.at[idx0.at[i, 0]], add=True)

        plsc.subcore_barrier()

        @pl.loop(0, ROWS_PER_TILE // 16)
        def _(j):
            rr = r0 + j * 16
            idxv = rr + iota
            pltpu.sync_copy(shared_acc.at[idxv], buf0.at[pl.ds(0, 16)])
            pltpu.sync_copy(buf0.at[pl.ds(0, 16)],
                            out_hbm.at[pl.ds(cid * N_PAD + rr, 16)])


def _scatter(msg, w, ridx, z):
    mesh = plsc.VectorSubcoreMesh(core_axis_name="c", subcore_axis_name="s")
    kern = pl.kernel(
        _scatter_kernel,
        mesh=mesh,
        out_type=[jax.ShapeDtypeStruct((2 * N_PAD, NODE_DIM), jnp.float32),
                  jax.ShapeDtypeStruct((2 * N_PAD, NODE_DIM), jnp.float32)],
        scratch_types=[
            pltpu.VMEM((SCHUNKS_PER_TILE, 1, SCHUNK), jnp.int32),
            pltpu.VMEM((SCHUNK, NODE_DIM), jnp.float32),
            pltpu.VMEM_SHARED((N_PAD, NODE_DIM), jnp.float32),
            pltpu.SemaphoreType.DMA,
        ],
    )
    return kern(msg, w, ridx.reshape(-1, 1, SCHUNK), z)


def _final_body(num, den, wo, bo, out):
    n = num[0] + num[1]
    d = den[0] + den[1]
    out[...] = jnp.dot(n * (1.0 / (d + 1e-8)), wo[...],
                       preferred_element_type=jnp.float32,
                       precision=_HI) + bo[...]


def _finalize(num_parts, den_parts, Wo, bo):
    blk = 1000
    grid = N_NODES // blk
    nb = pl.BlockSpec((2, blk, NODE_DIM), lambda i: (0, i, 0))
    return pl.pallas_call(
        _final_body,
        grid=(grid,),
        in_specs=[nb, nb,
                  pl.BlockSpec((NODE_DIM, NODE_DIM), lambda i: (0, 0)),
                  pl.BlockSpec((1, NODE_DIM), lambda i: (0, 0))],
        out_specs=pl.BlockSpec((blk, NODE_DIM), lambda i: (i, 0)),
        out_shape=jax.ShapeDtypeStruct((N_NODES, NODE_DIM), jnp.float32),
    )(num_parts, den_parts, Wo, bo.reshape(1, -1))


def kernel(nodes, edge_index, edge_attr, Wq, bq, Wk, bk, Wv, bv, We, be,
           Wo, bo):
    nodes_pad = jnp.pad(nodes, ((0, N_PAD - N_NODES), (0, 0)))
    q, k, v = _qkv(nodes_pad, Wq, bq, Wk, bk, Wv, bv)

    ea_pad = jnp.pad(edge_attr, ((0, E_PAD - N_EDGES), (0, 0)))

    row = edge_index[0].astype(jnp.int32)
    col = edge_index[1].astype(jnp.int32)
    pad_i = N_NODES + jnp.arange(E_PAD - N_EDGES, dtype=jnp.int32) % (
        N_PAD - N_NODES)
    ridx = jnp.concatenate([row, pad_i]).reshape(-1, CHUNK)
    cidx = jnp.concatenate([col, pad_i]).reshape(-1, CHUNK)

    qg, kg, vg = _gather(q, k, v, ridx, cidx)
    msg, w = _edge_math(qg, kg, vg, ea_pad, We, be)

    z = jnp.zeros((64, NODE_DIM), jnp.float32)
    num_parts, den_parts = _scatter(msg, w, ridx, z)

    return _finalize(num_parts.reshape(2, N_PAD, NODE_DIM),
                     den_parts.reshape(2, N_PAD, NODE_DIM), Wo, bo)
